# baseline (speedup 1.0000x reference)
import functools

import jax
import jax.numpy as jnp
from jax import lax
from jax.experimental import pallas as pl
from jax.experimental.pallas import tpu as pltpu
from jax.experimental.pallas import tpu_sc as plsc

B = 16384
N_COLS = 26
VOCAB = 100000
DIM = 16

NUM_CORES = 2
NUM_SUBCORES = 16
NW = NUM_CORES * NUM_SUBCORES
BT = B // 128
NTASK = N_COLS * BT
TPW = NTASK // NW
GPC = 8
CHUNK = GPC * 128
NCHUNK = TPW // GPC
HALF = GPC // 2
SPC = VOCAB // 8


def _gather_body(idx_hbm, tab_hbm, out_hbm, idx_v, sidx_v, q16_v, slabs_v,
                 tiles_v, sem):
  wid = lax.axis_index("s") * NUM_CORES + lax.axis_index("c")
  t0 = wid * TPW

  iota = lax.iota(jnp.int32, 16)

  def chunk_body(k, carry):
    t = t0 + k * GPC
    c = t // BT
    bt0 = pl.multiple_of(t - c * BT, GPC)
    pltpu.sync_copy(idx_hbm.at[c].at[pl.ds(bt0, GPC)], idx_v)
    for j in range(GPC):
      for l in range(8):
        v = idx_v[j, pl.ds(l * 16, 16)]
        sidx_v[j, pl.ds(l * 16, 16)] = lax.shift_right_logical(v, 3)
        q16_v[j, pl.ds(l * 16, 16)] = lax.shift_left(
            lax.bitwise_and(v, jnp.int32(7)), 4)
    for h in range(2):
      copies = []
      for jj in range(HALF):
        copies.append(
            pltpu.async_copy(
                tab_hbm.at[c].at[sidx_v.at[h * HALF + jj]],
                slabs_v.at[pl.ds(jj * 128, 128)], sem))
      for cp in copies:
        cp.wait()
      for jj in range(HALF):
        j = h * HALF + jj
        for l in range(8):
          q16s = q16_v[j, pl.ds(l * 16, 16)]
          ridx = iota + (jj * 128 + l * 16)
          for d in range(DIM):
            dt, di = d // 8, d % 8
            vals = plsc.load_gather(slabs_v, [ridx, q16s + d])
            tiles_v[dt, j, di, pl.ds(l * 16, 16)] = vals
    pltpu.sync_copy(tiles_v.at[0], out_hbm.at[c, 0].at[pl.ds(bt0, GPC)])
    pltpu.sync_copy(tiles_v.at[1], out_hbm.at[c, 1].at[pl.ds(bt0, GPC)])
    return carry

  lax.fori_loop(0, NCHUNK, chunk_body, 0)


@jax.jit
def _embed(idx3d, tab128):
  mesh = plsc.VectorSubcoreMesh(core_axis_name="c", subcore_axis_name="s")
  f = pl.kernel(
      _gather_body,
      mesh=mesh,
      out_type=jax.ShapeDtypeStruct((N_COLS, 2, BT, 8, 128), jnp.float32),
      scratch_types=[
          pltpu.VMEM((GPC, 128), jnp.int32),
          pltpu.VMEM((GPC, 128), jnp.int32),
          pltpu.VMEM((GPC, 128), jnp.int32),
          pltpu.VMEM((HALF * 128, 128), jnp.float32),
          pltpu.VMEM((2, GPC, 8, 128), jnp.float32),
          pltpu.SemaphoreType.DMA,
      ],
      compiler_params=pltpu.CompilerParams(
          use_tc_tiling_on_sc=True, needs_layout_passes=False),
  )
  return f(idx3d, tab128)


def kernel(value, tables):
  tab128 = tables.reshape(N_COLS, SPC, 128)
  idx3d = value.astype(jnp.int32).T.reshape(N_COLS, BT, 128)
  out5d = _embed(idx3d, tab128)
  return out5d.transpose(2, 4, 0, 1, 3).reshape(B, N_COLS, DIM)
